# baseline (speedup 1.0000x reference)
import functools

import jax
import jax.numpy as jnp
from jax import lax
from jax.experimental import pallas as pl
from jax.experimental.pallas import tpu as pltpu
from jax.experimental.pallas import tpu_sc as plsc

_N = 10000
_D = 256
_H = 128
_G = 64
_BN_EPS = 1e-5

_NC = 2
_NS = 16
_CH = 128
_NP = 10240
_ZR = 64
_BNR = 1000


def _fold_bn(lin, bn):
    s = bn["g"] / jnp.sqrt(bn["rv"] + _BN_EPS)
    return lin["W"] * s[None, :], (lin["b"] - bn["rm"]) * s + bn["be"]


def _mk_layer_in(first, interpret=False):
    grid = (_N // _BNR,)

    def body(*refs):
        if first:
            x_ref, wa_ref, ba_ref, oh_ref, hl_ref, r_ref, pooled_ref = refs
            hl = jnp.dot(x_ref[...], wa_ref[...],
                         preferred_element_type=jnp.float32) + ba_ref[...]
        else:
            h_ref, oh_ref, vn_ref, hl_ref, r_ref, pooled_ref = refs
            hl = h_ref[...] + jnp.dot(oh_ref[...], vn_ref[...],
                                      preferred_element_type=jnp.float32)
        i = pl.program_id(0)
        hl_ref[...] = hl
        r = jnp.maximum(hl, 0.0)
        r_ref[0] = r[:, :_H]
        r_ref[1] = r[:, _H:]
        contrib = lax.dot_general(oh_ref[...], hl, (((0,), (0,)), ((), ())),
                                  preferred_element_type=jnp.float32)

        @pl.when(i == 0)
        def _():
            pooled_ref[...] = contrib

        @pl.when(i > 0)
        def _():
            pooled_ref[...] += contrib

    if first:
        in_specs = [
            pl.BlockSpec((_BNR, _D), lambda i: (i, 0)),
            pl.BlockSpec((_D, _D), lambda i: (0, 0)),
            pl.BlockSpec((1, _D), lambda i: (0, 0)),
            pl.BlockSpec((_BNR, _G), lambda i: (i, 0)),
        ]
    else:
        in_specs = [
            pl.BlockSpec((_BNR, _D), lambda i: (i, 0)),
            pl.BlockSpec((_BNR, _G), lambda i: (i, 0)),
            pl.BlockSpec((_G, _D), lambda i: (0, 0)),
        ]
    return pl.pallas_call(
        body,
        grid=grid,
        in_specs=in_specs,
        out_specs=[
            pl.BlockSpec((_BNR, _D), lambda i: (i, 0)),
            pl.BlockSpec((2, _BNR, _H), lambda i: (0, i, 0)),
            pl.BlockSpec((_G, _D), lambda i: (0, 0)),
        ],
        out_shape=[
            jax.ShapeDtypeStruct((_N, _D), jnp.float32),
            jax.ShapeDtypeStruct((2, _N, _H), jnp.float32),
            jax.ShapeDtypeStruct((_G, _D), jnp.float32),
        ],
        compiler_params=pltpu.CompilerParams(dimension_semantics=("arbitrary",)),
        interpret=interpret,
    )


def _mk_gin_mlp(relu_out, interpret=False):
    grid = (_N // _BNR,)

    def body(eps_ref, hl_ref, agg_ref, w1_ref, b1_ref, w2_ref, b2_ref, out_ref):
        agg = jnp.concatenate([agg_ref[0], agg_ref[1]], axis=-1)
        t = eps_ref[0, 0] * hl_ref[...] + agg
        t = jnp.maximum(
            jnp.dot(t, w1_ref[...], preferred_element_type=jnp.float32)
            + b1_ref[...], 0.0)
        o = jnp.dot(t, w2_ref[...], preferred_element_type=jnp.float32) + b2_ref[...]
        if relu_out:
            o = jnp.maximum(o, 0.0)
        out_ref[...] = o

    return pl.pallas_call(
        body,
        grid=grid,
        in_specs=[
            pl.BlockSpec((1, 1), lambda i: (0, 0)),
            pl.BlockSpec((_BNR, _D), lambda i: (i, 0)),
            pl.BlockSpec((2, _BNR, _H), lambda i: (0, i, 0)),
            pl.BlockSpec((_D, _D), lambda i: (0, 0)),
            pl.BlockSpec((1, _D), lambda i: (0, 0)),
            pl.BlockSpec((_D, _D), lambda i: (0, 0)),
            pl.BlockSpec((1, _D), lambda i: (0, 0)),
        ],
        out_specs=pl.BlockSpec((_BNR, _D), lambda i: (i, 0)),
        out_shape=jax.ShapeDtypeStruct((_N, _D), jnp.float32),
        interpret=interpret,
    )


def _mk_vn_mlp(interpret=False):
    def body(p_ref, vn_ref, w1_ref, b1_ref, w2_ref, b2_ref, out_ref):
        vt = p_ref[...] + vn_ref[...]
        vt = jnp.maximum(
            jnp.dot(vt, w1_ref[...], preferred_element_type=jnp.float32)
            + b1_ref[...], 0.0)
        vt = jnp.maximum(
            jnp.dot(vt, w2_ref[...], preferred_element_type=jnp.float32)
            + b2_ref[...], 0.0)
        out_ref[...] = vt

    return pl.pallas_call(
        body,
        out_shape=jax.ShapeDtypeStruct((_G, _D), jnp.float32),
        interpret=interpret,
    )


_BLK = 16


def _mk_edge_agg(nblk):
    mesh = plsc.VectorSubcoreMesh(core_axis_name="c", subcore_axis_name="s",
                                  num_cores=_NC, num_subcores=_NS)
    zrows_per_tile = _NP // _NS

    @functools.partial(
        pl.kernel,
        out_type=jax.ShapeDtypeStruct((_NC, _NP, _H), jnp.float32),
        mesh=mesh,
        scratch_types=[
            pltpu.VMEM((2, _BLK, _CH), jnp.int32),
            pltpu.VMEM((2, _BLK, _CH), jnp.int32),
            pltpu.VMEM((2, _CH, _H), jnp.float32),
            pltpu.VMEM((_ZR, _H), jnp.float32),
            pltpu.VMEM_SHARED((_NP, _H), jnp.float32),
        ] + [pltpu.SemaphoreType.DMA] * 6,
    )
    def edge_agg(r_hbm, idx2_hbm, dst_hbm, out_hbm,
                 src_blk, dst_blk, buf_v, z_v, acc_sh, *sems):
        c = lax.axis_index("c")
        s = lax.axis_index("s")
        sg = sems[0:2]
        ss = sems[2:4]
        si = sems[4:6]

        zeros16 = jnp.zeros((16,), jnp.float32)

        def zfill(i, carry):
            for j in range(_H // 16):
                z_v[i, pl.ds(j * 16, 16)] = zeros16
            return carry

        lax.fori_loop(0, _ZR, zfill, 0)
        zbase = s * zrows_per_tile

        def zdma(k, carry):
            pltpu.sync_copy(z_v, acc_sh.at[pl.ds(zbase + k * _ZR, _ZR)])
            return carry

        lax.fori_loop(0, zrows_per_tile // _ZR, zdma, 0)

        def idx_load(abs_blk, p):
            pltpu.async_copy(idx2_hbm.at[c, abs_blk], src_blk.at[p], si[p])
            pltpu.async_copy(dst_hbm.at[abs_blk], dst_blk.at[p], si[p])

        def idx_wait(p):
            pltpu.make_async_copy(idx2_hbm.at[c, 0], src_blk.at[p],
                                  si[p]).wait()
            pltpu.make_async_copy(dst_hbm.at[0], dst_blk.at[p], si[p]).wait()

        def gather_start(p, m, q):
            pltpu.async_copy(r_hbm.at[src_blk.at[p, m]], buf_v.at[q], sg[q])

        def gather_wait(p, m, q):
            pltpu.make_async_copy(r_hbm.at[src_blk.at[p, m]], buf_v.at[q],
                                  sg[q]).wait()

        def scat_start(p, m, q):
            pltpu.async_copy(buf_v.at[q], acc_sh.at[dst_blk.at[p, m]], ss[q],
                             add=True)

        def scat_wait(p, m, q):
            pltpu.make_async_copy(buf_v.at[q], acc_sh.at[dst_blk.at[p, m]],
                                  ss[q]).wait()

        def emit_block(abs_next, p, first):
            for m in range(_BLK):
                q = m % 2
                pm, pp = (_BLK - 1, 1 - p) if m == 0 else (m - 1, p)
                pm2, pp2 = (_BLK + m - 2, 1 - p) if m <= 1 else (m - 2, p)
                if first and m <= 1:
                    gather_start(p, m, q)
                    if m == 1:
                        gather_wait(p, 0, 0)
                        scat_start(p, 0, 0)
                    continue
                scat_wait(pp2, pm2, q)
                gather_start(p, m, q)
                gather_wait(pp, pm, 1 - q)
                scat_start(pp, pm, 1 - q)
                if m == 2 and not first:
                    idx_load(abs_next, 1 - p)

        base_blk = s * nblk
        idx_load(base_blk, 0)
        idx_load(base_blk + 1, 1)
        idx_wait(0)
        plsc.subcore_barrier()
        emit_block(base_blk, 0, first=True)

        def body(j, carry):
            for r in range(2):
                cb = 1 + 2 * j + r
                p = (1 + r) % 2
                idx_wait(p)
                emit_block(base_blk + cb + 1, p, first=False)
            return carry

        lax.fori_loop(0, (nblk - 1) // 2, body, 0)

        idx_wait(1)
        gather_wait(0, _BLK - 1, 1)
        scat_start(0, _BLK - 1, 1)
        scat_wait(0, _BLK - 2, 0)
        scat_wait(0, _BLK - 1, 1)
        plsc.subcore_barrier()

        pltpu.sync_copy(acc_sh.at[pl.ds(zbase, zrows_per_tile)],
                        out_hbm.at[c, pl.ds(zbase, zrows_per_tile)])

    return edge_agg


def kernel(x, edge_index, batch, params):
    e = edge_index.shape[1]
    nck = -(-e // (_NS * _CH))
    nblk = -(-nck // _BLK)
    if nblk < 3:
        nblk = 3
    if nblk % 2 == 0:
        nblk += 1
    totb = _NS * nblk + 1
    epad = totb * _BLK * _CH
    src = edge_index[0].astype(jnp.int32)
    dst = edge_index[1].astype(jnp.int32)
    perm = jnp.argsort(src)
    src = src[perm]
    dst = dst[perm]
    srcp = jnp.concatenate([src, jnp.zeros((epad - e,), jnp.int32)])
    idx2 = jnp.stack([srcp, srcp + _N]).reshape(2, totb, _BLK, _CH)
    dstp = jnp.concatenate([dst, jnp.full((epad - e,), _N, jnp.int32)]
                           ).reshape(totb, _BLK, _CH)
    onehot = (batch[:, None] == jnp.arange(_G, dtype=batch.dtype)[None, :]
              ).astype(jnp.float32)

    layer_in0 = _mk_layer_in(first=True)
    layer_in = _mk_layer_in(first=False)
    edge_agg = _mk_edge_agg(nblk)
    gin_mid = _mk_gin_mlp(relu_out=True)
    gin_last = _mk_gin_mlp(relu_out=False)
    vn_mlp = _mk_vn_mlp()

    vn = jnp.broadcast_to(params["vn_emb"], (_G, _D))
    h = None
    for layer in range(3):
        cp = params["convs"][layer]
        if layer == 0:
            ba0 = (params["atom"]["b"] + params["vn_emb"][0]).reshape(1, _D)
            hl, r3, pooled = layer_in0(x, params["atom"]["W"], ba0, onehot)
        else:
            hl, r3, pooled = layer_in(h, onehot, vn)
        agg3 = edge_agg(r3.reshape(2 * _N, _H), idx2, dstp)
        w1, b1 = _fold_bn(cp["lin1"], cp["bn"])
        w2, b2 = _fold_bn(cp["lin2"], params["bns"][layer])
        epsp1 = (1.0 + cp["eps"]).reshape(1, 1)
        mlp = gin_mid if layer < 2 else gin_last
        h = mlp(epsp1, hl, agg3, w1, b1.reshape(1, _D), w2, b2.reshape(1, _D))
        if layer < 2:
            mp = params["vn_mlps"][layer]
            wv1, bv1 = _fold_bn(mp["lin1"], mp["bn1"])
            wv2, bv2 = _fold_bn(mp["lin2"], mp["bn2"])
            vn = vn_mlp(pooled, vn, wv1, bv1.reshape(1, _D),
                        wv2, bv2.reshape(1, _D))
    return h


# speedup vs baseline: 2.0587x; 2.0587x over previous
import functools

import jax
import jax.numpy as jnp
from jax import lax
from jax.experimental import pallas as pl
from jax.experimental.pallas import tpu as pltpu
from jax.experimental.pallas import tpu_sc as plsc

_N = 10000
_D = 256
_H = 128
_G = 64
_BN_EPS = 1e-5

_NC = 2
_NS = 16
_CH = 128
_NP = 10240
_ZR = 16
_BNR = 1000


def _fold_bn(lin, bn):
    s = bn["g"] / jnp.sqrt(bn["rv"] + _BN_EPS)
    return lin["W"] * s[None, :], (lin["b"] - bn["rm"]) * s + bn["be"]


def _mk_layer_in(first, interpret=False):
    grid = (_N // _BNR,)

    def body(*refs):
        if first:
            x_ref, wa_ref, ba_ref, oh_ref, hl_ref, r_ref, pooled_ref = refs
            hl = jnp.dot(x_ref[...], wa_ref[...],
                         preferred_element_type=jnp.float32) + ba_ref[...]
        else:
            h_ref, oh_ref, vn_ref, hl_ref, r_ref, pooled_ref = refs
            hl = h_ref[...] + jnp.dot(oh_ref[...], vn_ref[...],
                                      preferred_element_type=jnp.float32)
        i = pl.program_id(0)
        hl_ref[...] = hl
        r = jnp.maximum(hl, 0.0)
        r_ref[0] = r[:, :_H]
        r_ref[1] = r[:, _H:]
        contrib = lax.dot_general(oh_ref[...], hl, (((0,), (0,)), ((), ())),
                                  preferred_element_type=jnp.float32)

        @pl.when(i == 0)
        def _():
            pooled_ref[...] = contrib

        @pl.when(i > 0)
        def _():
            pooled_ref[...] += contrib

    if first:
        in_specs = [
            pl.BlockSpec((_BNR, _D), lambda i: (i, 0)),
            pl.BlockSpec((_D, _D), lambda i: (0, 0)),
            pl.BlockSpec((1, _D), lambda i: (0, 0)),
            pl.BlockSpec((_BNR, _G), lambda i: (i, 0)),
        ]
    else:
        in_specs = [
            pl.BlockSpec((_BNR, _D), lambda i: (i, 0)),
            pl.BlockSpec((_BNR, _G), lambda i: (i, 0)),
            pl.BlockSpec((_G, _D), lambda i: (0, 0)),
        ]
    return pl.pallas_call(
        body,
        grid=grid,
        in_specs=in_specs,
        out_specs=[
            pl.BlockSpec((_BNR, _D), lambda i: (i, 0)),
            pl.BlockSpec((2, _BNR, _H), lambda i: (0, i, 0)),
            pl.BlockSpec((_G, _D), lambda i: (0, 0)),
        ],
        out_shape=[
            jax.ShapeDtypeStruct((_N, _D), jnp.float32),
            jax.ShapeDtypeStruct((2, _NP, _H), jnp.float32),
            jax.ShapeDtypeStruct((_G, _D), jnp.float32),
        ],
        compiler_params=pltpu.CompilerParams(dimension_semantics=("arbitrary",)),
        interpret=interpret,
    )


def _mk_gin_mlp(relu_out, interpret=False):
    grid = (_N // _BNR,)

    def body(eps_ref, hl_ref, agg_ref, w1_ref, b1_ref, w2_ref, b2_ref, out_ref):
        agg = jnp.concatenate([agg_ref[0], agg_ref[1]], axis=-1)
        t = eps_ref[0, 0] * hl_ref[...] + agg
        t = jnp.maximum(
            jnp.dot(t, w1_ref[...], preferred_element_type=jnp.float32)
            + b1_ref[...], 0.0)
        o = jnp.dot(t, w2_ref[...], preferred_element_type=jnp.float32) + b2_ref[...]
        if relu_out:
            o = jnp.maximum(o, 0.0)
        out_ref[...] = o

    return pl.pallas_call(
        body,
        grid=grid,
        in_specs=[
            pl.BlockSpec((1, 1), lambda i: (0, 0)),
            pl.BlockSpec((_BNR, _D), lambda i: (i, 0)),
            pl.BlockSpec((2, _BNR, _H), lambda i: (0, i, 0)),
            pl.BlockSpec((_D, _D), lambda i: (0, 0)),
            pl.BlockSpec((1, _D), lambda i: (0, 0)),
            pl.BlockSpec((_D, _D), lambda i: (0, 0)),
            pl.BlockSpec((1, _D), lambda i: (0, 0)),
        ],
        out_specs=pl.BlockSpec((_BNR, _D), lambda i: (i, 0)),
        out_shape=jax.ShapeDtypeStruct((_N, _D), jnp.float32),
        interpret=interpret,
    )


def _mk_vn_mlp(interpret=False):
    def body(p_ref, vn_ref, w1_ref, b1_ref, w2_ref, b2_ref, out_ref):
        vt = p_ref[...] + vn_ref[...]
        vt = jnp.maximum(
            jnp.dot(vt, w1_ref[...], preferred_element_type=jnp.float32)
            + b1_ref[...], 0.0)
        vt = jnp.maximum(
            jnp.dot(vt, w2_ref[...], preferred_element_type=jnp.float32)
            + b2_ref[...], 0.0)
        out_ref[...] = vt

    return pl.pallas_call(
        body,
        out_shape=jax.ShapeDtypeStruct((_G, _D), jnp.float32),
        interpret=interpret,
    )


def _mk_edge_agg(nck):
    mesh = plsc.VectorSubcoreMesh(core_axis_name="c", subcore_axis_name="s",
                                  num_cores=_NC, num_subcores=_NS)
    zrows_per_tile = _NP // _NS
    srows = _NP // _NS
    rpt = nck * _CH

    @functools.partial(
        pl.kernel,
        out_type=[
            jax.ShapeDtypeStruct((_NC, _NP, _H), jnp.float32),
            jax.ShapeDtypeStruct((_NC, _NS * rpt, _H), jnp.float32),
        ],
        mesh=mesh,
        scratch_types=[
            pltpu.VMEM((nck, _CH), jnp.int32),
            pltpu.VMEM((2, _CH, _H), jnp.float32),
            pltpu.VMEM((_ZR, _H), jnp.float32),
            pltpu.VMEM_SHARED((_NP, _H), jnp.float32),
        ] + [pltpu.SemaphoreType.DMA] * 4,
    )
    def edge_agg(r_hbm, src_hbm, dst_hbm, out_hbm, scr_hbm,
                 idx_v, buf_v, z_v, sp, *sems):
        c = lax.axis_index("c")
        s = lax.axis_index("s")
        sa = sems[0:2]
        sb = sems[2:4]

        sbase = s * srows
        pltpu.sync_copy(r_hbm.at[pl.ds(c * _NP + sbase, srows)],
                        sp.at[pl.ds(sbase, srows)])
        pltpu.sync_copy(src_hbm.at[s], idx_v)
        plsc.subcore_barrier()

        rbase = s * rpt

        def g_start(k, q):
            pltpu.async_copy(sp.at[idx_v.at[k]], buf_v.at[q], sa[q])

        def g_wait(k, q):
            pltpu.make_async_copy(sp.at[idx_v.at[k]], buf_v.at[q],
                                  sa[q]).wait()

        def w_start(k, q):
            pltpu.async_copy(buf_v.at[q],
                             scr_hbm.at[c, pl.ds(rbase + k * _CH, _CH)],
                             sb[q])

        def w_wait(k, q):
            pltpu.make_async_copy(buf_v.at[q],
                                  scr_hbm.at[c, pl.ds(rbase + k * _CH, _CH)],
                                  sb[q]).wait()

        g_start(0, 0)
        g_start(1, 1)
        g_wait(0, 0)
        w_start(0, 0)

        def body_b(j, carry):
            for m in range(2):
                k = 2 * j + 2 + m
                q = m
                w_wait(k - 2, q)
                g_start(k, q)
                g_wait(k - 1, 1 - q)
                w_start(k - 1, 1 - q)
            return carry

        lax.fori_loop(0, (nck - 2) // 2, body_b, 0)
        g_wait(nck - 1, 1)
        w_start(nck - 1, 1)
        w_wait(nck - 2, 0)
        w_wait(nck - 1, 1)
        plsc.subcore_barrier()

        zeros16 = jnp.zeros((16,), jnp.float32)

        def zfill(i, carry):
            for j in range(_H // 16):
                z_v[i, pl.ds(j * 16, 16)] = zeros16
            return carry

        lax.fori_loop(0, _ZR, zfill, 0)
        zbase = s * zrows_per_tile

        def zdma(k, carry):
            pltpu.sync_copy(z_v, sp.at[pl.ds(zbase + k * _ZR, _ZR)])
            return carry

        lax.fori_loop(0, zrows_per_tile // _ZR, zdma, 0)
        pltpu.sync_copy(dst_hbm.at[s], idx_v)
        plsc.subcore_barrier()

        def r_start(k, q):
            pltpu.async_copy(scr_hbm.at[c, pl.ds(rbase + k * _CH, _CH)],
                             buf_v.at[q], sb[q])

        def r_wait(k, q):
            pltpu.make_async_copy(scr_hbm.at[c, pl.ds(rbase + k * _CH, _CH)],
                                  buf_v.at[q], sb[q]).wait()

        def a_start(k, q):
            pltpu.async_copy(buf_v.at[q], sp.at[idx_v.at[k]], sa[q], add=True)

        def a_wait(k, q):
            pltpu.make_async_copy(buf_v.at[q], sp.at[idx_v.at[k]],
                                  sa[q]).wait()

        r_start(0, 0)
        r_start(1, 1)
        r_wait(0, 0)
        a_start(0, 0)

        def body_d(j, carry):
            for m in range(2):
                k = 2 * j + 2 + m
                q = m
                a_wait(k - 2, q)
                r_start(k, q)
                r_wait(k - 1, 1 - q)
                a_start(k - 1, 1 - q)
            return carry

        lax.fori_loop(0, (nck - 2) // 2, body_d, 0)
        r_wait(nck - 1, 1)
        a_start(nck - 1, 1)
        a_wait(nck - 2, 0)
        a_wait(nck - 1, 1)
        plsc.subcore_barrier()

        pltpu.sync_copy(sp.at[pl.ds(zbase, zrows_per_tile)],
                        out_hbm.at[c, pl.ds(zbase, zrows_per_tile)])

    return edge_agg


def kernel(x, edge_index, batch, params):
    e = edge_index.shape[1]
    nck = -(-e // (_NS * _CH))
    if nck < 4:
        nck = 4
    if nck % 2:
        nck += 1
    epad = _NS * nck * _CH
    src = edge_index[0].astype(jnp.int32)
    dst = edge_index[1].astype(jnp.int32)
    srcp = jnp.concatenate([src, jnp.zeros((epad - e,), jnp.int32)]
                           ).reshape(_NS, nck, _CH)
    dstp = jnp.concatenate([dst, jnp.full((epad - e,), _N, jnp.int32)]
                           ).reshape(_NS, nck, _CH)
    onehot = (batch[:, None] == jnp.arange(_G, dtype=batch.dtype)[None, :]
              ).astype(jnp.float32)

    layer_in0 = _mk_layer_in(first=True)
    layer_in = _mk_layer_in(first=False)
    edge_agg = _mk_edge_agg(nck)
    gin_mid = _mk_gin_mlp(relu_out=True)
    gin_last = _mk_gin_mlp(relu_out=False)
    vn_mlp = _mk_vn_mlp()

    vn = jnp.broadcast_to(params["vn_emb"], (_G, _D))
    h = None
    for layer in range(3):
        cp = params["convs"][layer]
        if layer == 0:
            ba0 = (params["atom"]["b"] + params["vn_emb"][0]).reshape(1, _D)
            hl, r3, pooled = layer_in0(x, params["atom"]["W"], ba0, onehot)
        else:
            hl, r3, pooled = layer_in(h, onehot, vn)
        agg3, _ = edge_agg(r3.reshape(2 * _NP, _H), srcp, dstp)
        w1, b1 = _fold_bn(cp["lin1"], cp["bn"])
        w2, b2 = _fold_bn(cp["lin2"], params["bns"][layer])
        epsp1 = (1.0 + cp["eps"]).reshape(1, 1)
        mlp = gin_mid if layer < 2 else gin_last
        h = mlp(epsp1, hl, agg3, w1, b1.reshape(1, _D), w2, b2.reshape(1, _D))
        if layer < 2:
            mp = params["vn_mlps"][layer]
            wv1, bv1 = _fold_bn(mp["lin1"], mp["bn1"])
            wv2, bv2 = _fold_bn(mp["lin2"], mp["bn2"])
            vn = vn_mlp(pooled, vn, wv1, bv1.reshape(1, _D),
                        wv2, bv2.reshape(1, _D))
    return h


# speedup vs baseline: 2.1329x; 1.0361x over previous
import functools

import jax
import jax.numpy as jnp
from jax import lax
from jax.experimental import pallas as pl
from jax.experimental.pallas import tpu as pltpu
from jax.experimental.pallas import tpu_sc as plsc

_N = 10000
_D = 256
_H = 128
_G = 64
_BN_EPS = 1e-5

_NC = 2
_NS = 16
_CH = 128
_NP = 10240
_ZR = 16
_BNR = 1000


def _fold_bn(lin, bn):
    s = bn["g"] / jnp.sqrt(bn["rv"] + _BN_EPS)
    return lin["W"] * s[None, :], (lin["b"] - bn["rm"]) * s + bn["be"]


def _mk_layer_in(first, interpret=False):
    grid = (_N // _BNR,)

    def body(*refs):
        if first:
            x_ref, wa_ref, ba_ref, oh_ref, hl_ref, r_ref, pooled_ref = refs
            hl = jnp.dot(x_ref[...], wa_ref[...],
                         preferred_element_type=jnp.float32) + ba_ref[...]
        else:
            h_ref, oh_ref, vn_ref, hl_ref, r_ref, pooled_ref = refs
            hl = h_ref[...] + jnp.dot(oh_ref[...], vn_ref[...],
                                      preferred_element_type=jnp.float32)
        i = pl.program_id(0)
        hl_ref[...] = hl
        r = jnp.maximum(hl, 0.0)
        r_ref[0] = r[:, :_H]
        r_ref[1] = r[:, _H:]
        contrib = lax.dot_general(oh_ref[...], hl, (((0,), (0,)), ((), ())),
                                  preferred_element_type=jnp.float32)

        @pl.when(i == 0)
        def _():
            pooled_ref[...] = contrib

        @pl.when(i > 0)
        def _():
            pooled_ref[...] += contrib

    if first:
        in_specs = [
            pl.BlockSpec((_BNR, _D), lambda i: (i, 0)),
            pl.BlockSpec((_D, _D), lambda i: (0, 0)),
            pl.BlockSpec((1, _D), lambda i: (0, 0)),
            pl.BlockSpec((_BNR, _G), lambda i: (i, 0)),
        ]
    else:
        in_specs = [
            pl.BlockSpec((_BNR, _D), lambda i: (i, 0)),
            pl.BlockSpec((_BNR, _G), lambda i: (i, 0)),
            pl.BlockSpec((_G, _D), lambda i: (0, 0)),
        ]
    return pl.pallas_call(
        body,
        grid=grid,
        in_specs=in_specs,
        out_specs=[
            pl.BlockSpec((_BNR, _D), lambda i: (i, 0)),
            pl.BlockSpec((2, _BNR, _H), lambda i: (0, i, 0)),
            pl.BlockSpec((_G, _D), lambda i: (0, 0)),
        ],
        out_shape=[
            jax.ShapeDtypeStruct((_N, _D), jnp.float32),
            jax.ShapeDtypeStruct((2, _NP, _H), jnp.float32),
            jax.ShapeDtypeStruct((_G, _D), jnp.float32),
        ],
        compiler_params=pltpu.CompilerParams(dimension_semantics=("arbitrary",)),
        interpret=interpret,
    )


def _mk_gin_mlp(relu_out, interpret=False):
    grid = (_N // _BNR,)

    def body(eps_ref, hl_ref, agg_ref, w1_ref, b1_ref, w2_ref, b2_ref, out_ref):
        agg = jnp.concatenate([agg_ref[0], agg_ref[1]], axis=-1)
        t = eps_ref[0, 0] * hl_ref[...] + agg
        t = jnp.maximum(
            jnp.dot(t, w1_ref[...], preferred_element_type=jnp.float32)
            + b1_ref[...], 0.0)
        o = jnp.dot(t, w2_ref[...], preferred_element_type=jnp.float32) + b2_ref[...]
        if relu_out:
            o = jnp.maximum(o, 0.0)
        out_ref[...] = o

    return pl.pallas_call(
        body,
        grid=grid,
        in_specs=[
            pl.BlockSpec((1, 1), lambda i: (0, 0)),
            pl.BlockSpec((_BNR, _D), lambda i: (i, 0)),
            pl.BlockSpec((2, _BNR, _H), lambda i: (0, i, 0)),
            pl.BlockSpec((_D, _D), lambda i: (0, 0)),
            pl.BlockSpec((1, _D), lambda i: (0, 0)),
            pl.BlockSpec((_D, _D), lambda i: (0, 0)),
            pl.BlockSpec((1, _D), lambda i: (0, 0)),
        ],
        out_specs=pl.BlockSpec((_BNR, _D), lambda i: (i, 0)),
        out_shape=jax.ShapeDtypeStruct((_N, _D), jnp.float32),
        interpret=interpret,
    )


def _mk_gin_fused(interpret=False):
    grid = (_N // _BNR,)

    def body(eps_ref, hl_ref, agg_ref, w1_ref, b1_ref, w2_ref, b2_ref,
             oh_ref, vn_ref, hl2_ref, r_ref, pooled_ref):
        i = pl.program_id(0)
        agg = jnp.concatenate([agg_ref[0], agg_ref[1]], axis=-1)
        t = eps_ref[0, 0] * hl_ref[...] + agg
        t = jnp.maximum(
            jnp.dot(t, w1_ref[...], preferred_element_type=jnp.float32)
            + b1_ref[...], 0.0)
        h = jnp.maximum(
            jnp.dot(t, w2_ref[...], preferred_element_type=jnp.float32)
            + b2_ref[...], 0.0)
        hl = h + jnp.dot(oh_ref[...], vn_ref[...],
                         preferred_element_type=jnp.float32)
        hl2_ref[...] = hl
        r = jnp.maximum(hl, 0.0)
        r_ref[0] = r[:, :_H]
        r_ref[1] = r[:, _H:]
        contrib = lax.dot_general(oh_ref[...], hl, (((0,), (0,)), ((), ())),
                                  preferred_element_type=jnp.float32)

        @pl.when(i == 0)
        def _():
            pooled_ref[...] = contrib

        @pl.when(i > 0)
        def _():
            pooled_ref[...] += contrib

    return pl.pallas_call(
        body,
        grid=grid,
        in_specs=[
            pl.BlockSpec((1, 1), lambda i: (0, 0)),
            pl.BlockSpec((_BNR, _D), lambda i: (i, 0)),
            pl.BlockSpec((2, _BNR, _H), lambda i: (0, i, 0)),
            pl.BlockSpec((_D, _D), lambda i: (0, 0)),
            pl.BlockSpec((1, _D), lambda i: (0, 0)),
            pl.BlockSpec((_D, _D), lambda i: (0, 0)),
            pl.BlockSpec((1, _D), lambda i: (0, 0)),
            pl.BlockSpec((_BNR, _G), lambda i: (i, 0)),
            pl.BlockSpec((_G, _D), lambda i: (0, 0)),
        ],
        out_specs=[
            pl.BlockSpec((_BNR, _D), lambda i: (i, 0)),
            pl.BlockSpec((2, _BNR, _H), lambda i: (0, i, 0)),
            pl.BlockSpec((_G, _D), lambda i: (0, 0)),
        ],
        out_shape=[
            jax.ShapeDtypeStruct((_N, _D), jnp.float32),
            jax.ShapeDtypeStruct((2, _NP, _H), jnp.float32),
            jax.ShapeDtypeStruct((_G, _D), jnp.float32),
        ],
        compiler_params=pltpu.CompilerParams(dimension_semantics=("arbitrary",)),
        interpret=interpret,
    )


def _mk_vn_mlp(interpret=False):
    def body(p_ref, vn_ref, w1_ref, b1_ref, w2_ref, b2_ref, out_ref):
        vt = p_ref[...] + vn_ref[...]
        vt = jnp.maximum(
            jnp.dot(vt, w1_ref[...], preferred_element_type=jnp.float32)
            + b1_ref[...], 0.0)
        vt = jnp.maximum(
            jnp.dot(vt, w2_ref[...], preferred_element_type=jnp.float32)
            + b2_ref[...], 0.0)
        out_ref[...] = vt

    return pl.pallas_call(
        body,
        out_shape=jax.ShapeDtypeStruct((_G, _D), jnp.float32),
        interpret=interpret,
    )


def _mk_edge_agg(nck):
    mesh = plsc.VectorSubcoreMesh(core_axis_name="c", subcore_axis_name="s",
                                  num_cores=_NC, num_subcores=_NS)
    zrows_per_tile = _NP // _NS
    srows = _NP // _NS
    rpt = nck * _CH

    @functools.partial(
        pl.kernel,
        out_type=[
            jax.ShapeDtypeStruct((_NC, _NP, _H), jnp.float32),
            jax.ShapeDtypeStruct((_NC, _NS * rpt, _H), jnp.float32),
        ],
        mesh=mesh,
        scratch_types=[
            pltpu.VMEM((nck, _CH), jnp.int32),
            pltpu.VMEM((2, _CH, _H), jnp.float32),
            pltpu.VMEM((_ZR, _H), jnp.float32),
            pltpu.VMEM_SHARED((_NP, _H), jnp.float32),
        ] + [pltpu.SemaphoreType.DMA] * 4,
    )
    def edge_agg(r_hbm, src_hbm, dst_hbm, out_hbm, scr_hbm,
                 idx_v, buf_v, z_v, sp, *sems):
        c = lax.axis_index("c")
        s = lax.axis_index("s")
        sa = sems[0:2]
        sb = sems[2:4]

        sbase = s * srows
        pltpu.sync_copy(r_hbm.at[pl.ds(c * _NP + sbase, srows)],
                        sp.at[pl.ds(sbase, srows)])
        pltpu.sync_copy(src_hbm.at[s], idx_v)
        plsc.subcore_barrier()

        rbase = s * rpt

        def g_start(k, q):
            pltpu.async_copy(sp.at[idx_v.at[k]], buf_v.at[q], sa[q])

        def g_wait(k, q):
            pltpu.make_async_copy(sp.at[idx_v.at[k]], buf_v.at[q],
                                  sa[q]).wait()

        def w_start(k, q):
            pltpu.async_copy(buf_v.at[q],
                             scr_hbm.at[c, pl.ds(rbase + k * _CH, _CH)],
                             sb[q])

        def w_wait(k, q):
            pltpu.make_async_copy(buf_v.at[q],
                                  scr_hbm.at[c, pl.ds(rbase + k * _CH, _CH)],
                                  sb[q]).wait()

        g_start(0, 0)
        g_start(1, 1)
        g_wait(0, 0)
        w_start(0, 0)

        def body_b(j, carry):
            for m in range(2):
                k = 2 * j + 2 + m
                q = m
                w_wait(k - 2, q)
                g_start(k, q)
                g_wait(k - 1, 1 - q)
                w_start(k - 1, 1 - q)
            return carry

        lax.fori_loop(0, (nck - 2) // 2, body_b, 0)
        g_wait(nck - 1, 1)
        w_start(nck - 1, 1)
        w_wait(nck - 2, 0)
        w_wait(nck - 1, 1)
        plsc.subcore_barrier()

        zeros16 = jnp.zeros((16,), jnp.float32)

        def zfill(i, carry):
            for j in range(_H // 16):
                z_v[i, pl.ds(j * 16, 16)] = zeros16
            return carry

        lax.fori_loop(0, _ZR, zfill, 0)
        zbase = s * zrows_per_tile

        def zdma(k, carry):
            pltpu.sync_copy(z_v, sp.at[pl.ds(zbase + k * _ZR, _ZR)])
            return carry

        lax.fori_loop(0, zrows_per_tile // _ZR, zdma, 0)
        pltpu.sync_copy(dst_hbm.at[s], idx_v)
        plsc.subcore_barrier()

        def r_start(k, q):
            pltpu.async_copy(scr_hbm.at[c, pl.ds(rbase + k * _CH, _CH)],
                             buf_v.at[q], sb[q])

        def r_wait(k, q):
            pltpu.make_async_copy(scr_hbm.at[c, pl.ds(rbase + k * _CH, _CH)],
                                  buf_v.at[q], sb[q]).wait()

        def a_start(k, q):
            pltpu.async_copy(buf_v.at[q], sp.at[idx_v.at[k]], sa[q], add=True)

        def a_wait(k, q):
            pltpu.make_async_copy(buf_v.at[q], sp.at[idx_v.at[k]],
                                  sa[q]).wait()

        r_start(0, 0)
        r_start(1, 1)
        r_wait(0, 0)
        a_start(0, 0)

        def body_d(j, carry):
            for m in range(2):
                k = 2 * j + 2 + m
                q = m
                a_wait(k - 2, q)
                r_start(k, q)
                r_wait(k - 1, 1 - q)
                a_start(k - 1, 1 - q)
            return carry

        lax.fori_loop(0, (nck - 2) // 2, body_d, 0)
        r_wait(nck - 1, 1)
        a_start(nck - 1, 1)
        a_wait(nck - 2, 0)
        a_wait(nck - 1, 1)
        plsc.subcore_barrier()

        pltpu.sync_copy(sp.at[pl.ds(zbase, zrows_per_tile)],
                        out_hbm.at[c, pl.ds(zbase, zrows_per_tile)])

    return edge_agg


def kernel(x, edge_index, batch, params):
    e = edge_index.shape[1]
    nck = -(-e // (_NS * _CH))
    if nck < 4:
        nck = 4
    if nck % 2:
        nck += 1
    epad = _NS * nck * _CH
    src = edge_index[0].astype(jnp.int32)
    dst = edge_index[1].astype(jnp.int32)
    srcp = jnp.concatenate([src, jnp.zeros((epad - e,), jnp.int32)]
                           ).reshape(_NS, nck, _CH)
    dstp = jnp.concatenate([dst, jnp.full((epad - e,), _N, jnp.int32)]
                           ).reshape(_NS, nck, _CH)
    onehot = (batch[:, None] == jnp.arange(_G, dtype=batch.dtype)[None, :]
              ).astype(jnp.float32)

    layer_in0 = _mk_layer_in(first=True)
    edge_agg = _mk_edge_agg(nck)
    gin_fused = _mk_gin_fused()
    gin_last = _mk_gin_mlp(relu_out=False)
    vn_mlp = _mk_vn_mlp()

    def layer_params(layer):
        cp = params["convs"][layer]
        w1, b1 = _fold_bn(cp["lin1"], cp["bn"])
        w2, b2 = _fold_bn(cp["lin2"], params["bns"][layer])
        epsp1 = (1.0 + cp["eps"]).reshape(1, 1)
        return epsp1, w1, b1.reshape(1, _D), w2, b2.reshape(1, _D)

    def vn_update(layer, pooled, vn):
        mp = params["vn_mlps"][layer]
        wv1, bv1 = _fold_bn(mp["lin1"], mp["bn1"])
        wv2, bv2 = _fold_bn(mp["lin2"], mp["bn2"])
        return vn_mlp(pooled, vn, wv1, bv1.reshape(1, _D),
                      wv2, bv2.reshape(1, _D))

    vn = jnp.broadcast_to(params["vn_emb"], (_G, _D))
    ba0 = (params["atom"]["b"] + params["vn_emb"][0]).reshape(1, _D)
    hl, r3, pooled = layer_in0(x, params["atom"]["W"], ba0, onehot)
    for layer in range(2):
        vn = vn_update(layer, pooled, vn)
        agg3, _ = edge_agg(r3.reshape(2 * _NP, _H), srcp, dstp)
        epsp1, w1, b1, w2, b2 = layer_params(layer)
        hl, r3, pooled = gin_fused(epsp1, hl, agg3, w1, b1, w2, b2,
                                   onehot, vn)
    agg3, _ = edge_agg(r3.reshape(2 * _NP, _H), srcp, dstp)
    epsp1, w1, b1, w2, b2 = layer_params(2)
    return gin_last(epsp1, hl, agg3, w1, b1, w2, b2)


# speedup vs baseline: 2.1405x; 1.0035x over previous
import functools

import jax
import jax.numpy as jnp
from jax import lax
from jax.experimental import pallas as pl
from jax.experimental.pallas import tpu as pltpu
from jax.experimental.pallas import tpu_sc as plsc

_N = 10000
_D = 256
_H = 128
_G = 64
_BN_EPS = 1e-5

_NC = 2
_NS = 16
_CH = 128
_NP = 10240
_ZR = 32
_BNR = 1000


def _fold_bn(lin, bn):
    s = bn["g"] / jnp.sqrt(bn["rv"] + _BN_EPS)
    return lin["W"] * s[None, :], (lin["b"] - bn["rm"]) * s + bn["be"]


def _mk_layer_in(first, interpret=False):
    grid = (_N // _BNR,)

    def body(*refs):
        if first:
            x_ref, wa_ref, ba_ref, oh_ref, hl_ref, r_ref, pooled_ref = refs
            hl = jnp.dot(x_ref[...], wa_ref[...],
                         preferred_element_type=jnp.float32) + ba_ref[...]
        else:
            h_ref, oh_ref, vn_ref, hl_ref, r_ref, pooled_ref = refs
            hl = h_ref[...] + jnp.dot(oh_ref[...], vn_ref[...],
                                      preferred_element_type=jnp.float32)
        i = pl.program_id(0)
        hl_ref[...] = hl
        r = jnp.maximum(hl, 0.0)
        r_ref[0] = r[:, :_H]
        r_ref[1] = r[:, _H:]
        contrib = lax.dot_general(oh_ref[...], hl, (((0,), (0,)), ((), ())),
                                  preferred_element_type=jnp.float32)

        @pl.when(i == 0)
        def _():
            pooled_ref[...] = contrib

        @pl.when(i > 0)
        def _():
            pooled_ref[...] += contrib

    if first:
        in_specs = [
            pl.BlockSpec((_BNR, _D), lambda i: (i, 0)),
            pl.BlockSpec((_D, _D), lambda i: (0, 0)),
            pl.BlockSpec((1, _D), lambda i: (0, 0)),
            pl.BlockSpec((_BNR, _G), lambda i: (i, 0)),
        ]
    else:
        in_specs = [
            pl.BlockSpec((_BNR, _D), lambda i: (i, 0)),
            pl.BlockSpec((_BNR, _G), lambda i: (i, 0)),
            pl.BlockSpec((_G, _D), lambda i: (0, 0)),
        ]
    return pl.pallas_call(
        body,
        grid=grid,
        in_specs=in_specs,
        out_specs=[
            pl.BlockSpec((_BNR, _D), lambda i: (i, 0)),
            pl.BlockSpec((2, _BNR, _H), lambda i: (0, i, 0)),
            pl.BlockSpec((_G, _D), lambda i: (0, 0)),
        ],
        out_shape=[
            jax.ShapeDtypeStruct((_N, _D), jnp.float32),
            jax.ShapeDtypeStruct((2, _NP, _H), jnp.float32),
            jax.ShapeDtypeStruct((_G, _D), jnp.float32),
        ],
        compiler_params=pltpu.CompilerParams(dimension_semantics=("arbitrary",)),
        interpret=interpret,
    )


def _mk_gin_mlp(relu_out, interpret=False):
    grid = (_N // _BNR,)

    def body(eps_ref, hl_ref, agg_ref, w1_ref, b1_ref, w2_ref, b2_ref, out_ref):
        agg = jnp.concatenate([agg_ref[0], agg_ref[1]], axis=-1)
        t = eps_ref[0, 0] * hl_ref[...] + agg
        t = jnp.maximum(
            jnp.dot(t, w1_ref[...], preferred_element_type=jnp.float32)
            + b1_ref[...], 0.0)
        o = jnp.dot(t, w2_ref[...], preferred_element_type=jnp.float32) + b2_ref[...]
        if relu_out:
            o = jnp.maximum(o, 0.0)
        out_ref[...] = o

    return pl.pallas_call(
        body,
        grid=grid,
        in_specs=[
            pl.BlockSpec((1, 1), lambda i: (0, 0)),
            pl.BlockSpec((_BNR, _D), lambda i: (i, 0)),
            pl.BlockSpec((2, _BNR, _H), lambda i: (0, i, 0)),
            pl.BlockSpec((_D, _D), lambda i: (0, 0)),
            pl.BlockSpec((1, _D), lambda i: (0, 0)),
            pl.BlockSpec((_D, _D), lambda i: (0, 0)),
            pl.BlockSpec((1, _D), lambda i: (0, 0)),
        ],
        out_specs=pl.BlockSpec((_BNR, _D), lambda i: (i, 0)),
        out_shape=jax.ShapeDtypeStruct((_N, _D), jnp.float32),
        interpret=interpret,
    )


def _mk_gin_fused(interpret=False):
    grid = (_N // _BNR,)

    def body(eps_ref, hl_ref, agg_ref, w1_ref, b1_ref, w2_ref, b2_ref,
             oh_ref, vn_ref, hl2_ref, r_ref, pooled_ref):
        i = pl.program_id(0)
        agg = jnp.concatenate([agg_ref[0], agg_ref[1]], axis=-1)
        t = eps_ref[0, 0] * hl_ref[...] + agg
        t = jnp.maximum(
            jnp.dot(t, w1_ref[...], preferred_element_type=jnp.float32)
            + b1_ref[...], 0.0)
        h = jnp.maximum(
            jnp.dot(t, w2_ref[...], preferred_element_type=jnp.float32)
            + b2_ref[...], 0.0)
        hl = h + jnp.dot(oh_ref[...], vn_ref[...],
                         preferred_element_type=jnp.float32)
        hl2_ref[...] = hl
        r = jnp.maximum(hl, 0.0)
        r_ref[0] = r[:, :_H]
        r_ref[1] = r[:, _H:]
        contrib = lax.dot_general(oh_ref[...], hl, (((0,), (0,)), ((), ())),
                                  preferred_element_type=jnp.float32)

        @pl.when(i == 0)
        def _():
            pooled_ref[...] = contrib

        @pl.when(i > 0)
        def _():
            pooled_ref[...] += contrib

    return pl.pallas_call(
        body,
        grid=grid,
        in_specs=[
            pl.BlockSpec((1, 1), lambda i: (0, 0)),
            pl.BlockSpec((_BNR, _D), lambda i: (i, 0)),
            pl.BlockSpec((2, _BNR, _H), lambda i: (0, i, 0)),
            pl.BlockSpec((_D, _D), lambda i: (0, 0)),
            pl.BlockSpec((1, _D), lambda i: (0, 0)),
            pl.BlockSpec((_D, _D), lambda i: (0, 0)),
            pl.BlockSpec((1, _D), lambda i: (0, 0)),
            pl.BlockSpec((_BNR, _G), lambda i: (i, 0)),
            pl.BlockSpec((_G, _D), lambda i: (0, 0)),
        ],
        out_specs=[
            pl.BlockSpec((_BNR, _D), lambda i: (i, 0)),
            pl.BlockSpec((2, _BNR, _H), lambda i: (0, i, 0)),
            pl.BlockSpec((_G, _D), lambda i: (0, 0)),
        ],
        out_shape=[
            jax.ShapeDtypeStruct((_N, _D), jnp.float32),
            jax.ShapeDtypeStruct((2, _NP, _H), jnp.float32),
            jax.ShapeDtypeStruct((_G, _D), jnp.float32),
        ],
        compiler_params=pltpu.CompilerParams(dimension_semantics=("arbitrary",)),
        interpret=interpret,
    )


def _mk_vn_mlp(interpret=False):
    def body(p_ref, vn_ref, w1_ref, b1_ref, w2_ref, b2_ref, out_ref):
        vt = p_ref[...] + vn_ref[...]
        vt = jnp.maximum(
            jnp.dot(vt, w1_ref[...], preferred_element_type=jnp.float32)
            + b1_ref[...], 0.0)
        vt = jnp.maximum(
            jnp.dot(vt, w2_ref[...], preferred_element_type=jnp.float32)
            + b2_ref[...], 0.0)
        out_ref[...] = vt

    return pl.pallas_call(
        body,
        out_shape=jax.ShapeDtypeStruct((_G, _D), jnp.float32),
        interpret=interpret,
    )


def _mk_edge_agg(nck):
    mesh = plsc.VectorSubcoreMesh(core_axis_name="c", subcore_axis_name="s",
                                  num_cores=_NC, num_subcores=_NS)
    zrows_per_tile = _NP // _NS
    srows = _NP // _NS
    rpt = nck * _CH

    @functools.partial(
        pl.kernel,
        out_type=[
            jax.ShapeDtypeStruct((_NC, _NP, _H), jnp.float32),
            jax.ShapeDtypeStruct((_NC, _NS * rpt, _H), jnp.float32),
        ],
        mesh=mesh,
        scratch_types=[
            pltpu.VMEM((nck, _CH), jnp.int32),
            pltpu.VMEM((2, _CH, _H), jnp.float32),
            pltpu.VMEM((_ZR, _H), jnp.float32),
            pltpu.VMEM_SHARED((_NP, _H), jnp.float32),
        ] + [pltpu.SemaphoreType.DMA] * 4,
    )
    def edge_agg(r_hbm, src_hbm, dst_hbm, out_hbm, scr_hbm,
                 idx_v, buf_v, z_v, sp, *sems):
        c = lax.axis_index("c")
        s = lax.axis_index("s")
        sa = sems[0:2]
        sb = sems[2:4]

        sbase = s * srows
        pltpu.sync_copy(r_hbm.at[pl.ds(c * _NP + sbase, srows)],
                        sp.at[pl.ds(sbase, srows)])
        pltpu.sync_copy(src_hbm.at[s], idx_v)
        zeros16 = jnp.zeros((16,), jnp.float32)

        def zfill(i, carry):
            for j in range(_H // 16):
                z_v[i, pl.ds(j * 16, 16)] = zeros16
            return carry

        lax.fori_loop(0, _ZR, zfill, 0)
        plsc.subcore_barrier()

        rbase = s * rpt

        def g_start(k, q):
            pltpu.async_copy(sp.at[idx_v.at[k]], buf_v.at[q], sa[q])

        def g_wait(k, q):
            pltpu.make_async_copy(sp.at[idx_v.at[k]], buf_v.at[q],
                                  sa[q]).wait()

        def w_start(k, q):
            pltpu.async_copy(buf_v.at[q],
                             scr_hbm.at[c, pl.ds(rbase + k * _CH, _CH)],
                             sb[q])

        def w_wait(k, q):
            pltpu.make_async_copy(buf_v.at[q],
                                  scr_hbm.at[c, pl.ds(rbase + k * _CH, _CH)],
                                  sb[q]).wait()

        g_start(0, 0)
        g_start(1, 1)
        g_wait(0, 0)
        w_start(0, 0)

        def body_b(j, carry):
            for m in range(2):
                k = 2 * j + 2 + m
                q = m
                w_wait(k - 2, q)
                g_start(k, q)
                g_wait(k - 1, 1 - q)
                w_start(k - 1, 1 - q)
            return carry

        lax.fori_loop(0, (nck - 2) // 2, body_b, 0)
        g_wait(nck - 1, 1)
        w_start(nck - 1, 1)
        w_wait(nck - 2, 0)
        w_wait(nck - 1, 1)
        plsc.subcore_barrier()

        zbase = s * zrows_per_tile

        def zdma(k, carry):
            pltpu.sync_copy(z_v, sp.at[pl.ds(zbase + k * _ZR, _ZR)])
            return carry

        lax.fori_loop(0, zrows_per_tile // _ZR, zdma, 0)
        pltpu.sync_copy(dst_hbm.at[s], idx_v)
        plsc.subcore_barrier()

        def r_start(k, q):
            pltpu.async_copy(scr_hbm.at[c, pl.ds(rbase + k * _CH, _CH)],
                             buf_v.at[q], sb[q])

        def r_wait(k, q):
            pltpu.make_async_copy(scr_hbm.at[c, pl.ds(rbase + k * _CH, _CH)],
                                  buf_v.at[q], sb[q]).wait()

        def a_start(k, q):
            pltpu.async_copy(buf_v.at[q], sp.at[idx_v.at[k]], sa[q], add=True)

        def a_wait(k, q):
            pltpu.make_async_copy(buf_v.at[q], sp.at[idx_v.at[k]],
                                  sa[q]).wait()

        r_start(0, 0)
        r_start(1, 1)
        r_wait(0, 0)
        a_start(0, 0)

        def body_d(j, carry):
            for m in range(2):
                k = 2 * j + 2 + m
                q = m
                a_wait(k - 2, q)
                r_start(k, q)
                r_wait(k - 1, 1 - q)
                a_start(k - 1, 1 - q)
            return carry

        lax.fori_loop(0, (nck - 2) // 2, body_d, 0)
        r_wait(nck - 1, 1)
        a_start(nck - 1, 1)
        a_wait(nck - 2, 0)
        a_wait(nck - 1, 1)
        plsc.subcore_barrier()

        pltpu.sync_copy(sp.at[pl.ds(zbase, zrows_per_tile)],
                        out_hbm.at[c, pl.ds(zbase, zrows_per_tile)])

    return edge_agg


def kernel(x, edge_index, batch, params):
    e = edge_index.shape[1]
    nck = -(-e // (_NS * _CH))
    if nck < 4:
        nck = 4
    if nck % 2:
        nck += 1
    epad = _NS * nck * _CH
    src = edge_index[0].astype(jnp.int32)
    dst = edge_index[1].astype(jnp.int32)
    srcp = jnp.concatenate([src, jnp.zeros((epad - e,), jnp.int32)]
                           ).reshape(_NS, nck, _CH)
    dstp = jnp.concatenate([dst, jnp.full((epad - e,), _N, jnp.int32)]
                           ).reshape(_NS, nck, _CH)
    onehot = (batch[:, None] == jnp.arange(_G, dtype=batch.dtype)[None, :]
              ).astype(jnp.float32)

    layer_in0 = _mk_layer_in(first=True)
    edge_agg = _mk_edge_agg(nck)
    gin_fused = _mk_gin_fused()
    gin_last = _mk_gin_mlp(relu_out=False)
    vn_mlp = _mk_vn_mlp()

    def layer_params(layer):
        cp = params["convs"][layer]
        w1, b1 = _fold_bn(cp["lin1"], cp["bn"])
        w2, b2 = _fold_bn(cp["lin2"], params["bns"][layer])
        epsp1 = (1.0 + cp["eps"]).reshape(1, 1)
        return epsp1, w1, b1.reshape(1, _D), w2, b2.reshape(1, _D)

    def vn_update(layer, pooled, vn):
        mp = params["vn_mlps"][layer]
        wv1, bv1 = _fold_bn(mp["lin1"], mp["bn1"])
        wv2, bv2 = _fold_bn(mp["lin2"], mp["bn2"])
        return vn_mlp(pooled, vn, wv1, bv1.reshape(1, _D),
                      wv2, bv2.reshape(1, _D))

    vn = jnp.broadcast_to(params["vn_emb"], (_G, _D))
    ba0 = (params["atom"]["b"] + params["vn_emb"][0]).reshape(1, _D)
    hl, r3, pooled = layer_in0(x, params["atom"]["W"], ba0, onehot)
    for layer in range(2):
        vn = vn_update(layer, pooled, vn)
        agg3, _ = edge_agg(r3.reshape(2 * _NP, _H), srcp, dstp)
        epsp1, w1, b1, w2, b2 = layer_params(layer)
        hl, r3, pooled = gin_fused(epsp1, hl, agg3, w1, b1, w2, b2,
                                   onehot, vn)
    agg3, _ = edge_agg(r3.reshape(2 * _NP, _H), srcp, dstp)
    epsp1, w1, b1, w2, b2 = layer_params(2)
    return gin_last(epsp1, hl, agg3, w1, b1, w2, b2)
